# baseline (speedup 1.0000x reference)
import jax
import jax.numpy as jnp
from jax.experimental import pallas as pl
from jax.experimental.pallas import tpu as pltpu

_DIM = 2048
_N_EXPERTS = 16
_TOKENS = 16384
_BLOCK_T = 16384


def _tail_block(s_ref, w_out_ref, i_out_ref):
    st = jnp.transpose(s_ref[...])
    iota = jax.lax.broadcasted_iota(jnp.int32, st.shape, 0).astype(jnp.float32)
    m = jnp.max(st, axis=0, keepdims=True)
    e = jnp.exp(st - m)
    p = e / jnp.sum(e, axis=0, keepdims=True)
    v1 = jnp.max(p, axis=0, keepdims=True)
    i1 = jnp.min(jnp.where(p == v1, iota, float(_N_EXPERTS)),
                 axis=0, keepdims=True)
    p2 = jnp.where(iota == i1, -1.0, p)
    v2 = jnp.max(p2, axis=0, keepdims=True)
    i2 = jnp.min(jnp.where(p2 == v2, iota, float(_N_EXPERTS)),
                 axis=0, keepdims=True)
    s = v1 + v2
    w2t = jnp.concatenate([v1 / s, v2 / s], axis=0)
    i2t = jnp.concatenate([i1, i2], axis=0).astype(jnp.int32)
    w_out_ref[...] = jnp.transpose(w2t)
    i_out_ref[...] = jnp.transpose(i2t)


def kernel(x, W, b):
    scores = x @ W.T + b
    grid = (_TOKENS // _BLOCK_T,)
    weights, indices = pl.pallas_call(
        _tail_block,
        grid=grid,
        in_specs=[pl.BlockSpec((_BLOCK_T, _N_EXPERTS), lambda i: (i, 0))],
        out_specs=[
            pl.BlockSpec((_BLOCK_T, 2), lambda i: (i, 0)),
            pl.BlockSpec((_BLOCK_T, 2), lambda i: (i, 0)),
        ],
        out_shape=[
            jax.ShapeDtypeStruct((_TOKENS, 2), jnp.float32),
            jax.ShapeDtypeStruct((_TOKENS, 2), jnp.int32),
        ],
        compiler_params=pltpu.CompilerParams(
            dimension_semantics=("arbitrary",),
        ),
    )(scores)
    return (weights, indices)


# speedup vs baseline: 1.1777x; 1.1777x over previous
import jax
import jax.numpy as jnp
from jax.experimental import pallas as pl
from jax.experimental.pallas import tpu as pltpu

_DIM = 2048
_N_EXPERTS = 16
_TOKENS = 16384
_BLOCK_T = 2048


def _gate_block(x_ref, w_ref, b_ref, w_out_ref, i_out_ref):
    st = jax.lax.dot_general(
        w_ref[...], x_ref[...],
        dimension_numbers=(((1,), (1,)), ((), ())),
        preferred_element_type=jnp.float32,
    ) + b_ref[...]
    iota = jax.lax.broadcasted_iota(jnp.int32, st.shape, 0).astype(jnp.float32)
    m = jnp.max(st, axis=0, keepdims=True)
    e = jnp.exp(st - m)
    p = e / jnp.sum(e, axis=0, keepdims=True)
    v1 = jnp.max(p, axis=0, keepdims=True)
    i1 = jnp.min(jnp.where(p == v1, iota, float(_N_EXPERTS)),
                 axis=0, keepdims=True)
    p2 = jnp.where(iota == i1, -1.0, p)
    v2 = jnp.max(p2, axis=0, keepdims=True)
    i2 = jnp.min(jnp.where(p2 == v2, iota, float(_N_EXPERTS)),
                 axis=0, keepdims=True)
    s = v1 + v2
    w2t = jnp.concatenate([v1 / s, v2 / s], axis=0)
    i2t = jnp.concatenate([i1, i2], axis=0).astype(jnp.int32)
    w_out_ref[...] = jnp.transpose(w2t)
    i_out_ref[...] = jnp.transpose(i2t)


def kernel(x, W, b):
    b2 = b.reshape(_N_EXPERTS, 1)
    grid = (_TOKENS // _BLOCK_T,)
    weights, indices = pl.pallas_call(
        _gate_block,
        grid=grid,
        in_specs=[
            pl.BlockSpec((_BLOCK_T, _DIM), lambda i: (i, 0)),
            pl.BlockSpec((_N_EXPERTS, _DIM), lambda i: (0, 0)),
            pl.BlockSpec((_N_EXPERTS, 1), lambda i: (0, 0)),
        ],
        out_specs=[
            pl.BlockSpec((_BLOCK_T, 2), lambda i: (i, 0)),
            pl.BlockSpec((_BLOCK_T, 2), lambda i: (i, 0)),
        ],
        out_shape=[
            jax.ShapeDtypeStruct((_TOKENS, 2), jnp.float32),
            jax.ShapeDtypeStruct((_TOKENS, 2), jnp.int32),
        ],
        compiler_params=pltpu.CompilerParams(
            dimension_semantics=("arbitrary",),
        ),
    )(x, W, b2)
    return (weights, indices)


# speedup vs baseline: 1.1880x; 1.0088x over previous
import jax
import jax.numpy as jnp
from jax.experimental import pallas as pl
from jax.experimental.pallas import tpu as pltpu

_DIM = 2048
_N_EXPERTS = 16
_TOKENS = 16384
_BLOCK_T = 1024


def _gate_one(x_ref, w_ref, b_ref, w_out_ref, i_out_ref):
    st = jax.lax.dot_general(
        w_ref[...], x_ref[...],
        dimension_numbers=(((1,), (1,)), ((), ())),
        preferred_element_type=jnp.float32,
    ) + b_ref[...]
    iota = jax.lax.broadcasted_iota(jnp.int32, st.shape, 0).astype(jnp.float32)
    m = jnp.max(st, axis=0, keepdims=True)
    e = jnp.exp(st - m)
    p = e / jnp.sum(e, axis=0, keepdims=True)
    v1 = jnp.max(p, axis=0, keepdims=True)
    i1 = jnp.min(jnp.where(p == v1, iota, float(_N_EXPERTS)),
                 axis=0, keepdims=True)
    p2 = jnp.where(iota == i1, -1.0, p)
    v2 = jnp.max(p2, axis=0, keepdims=True)
    i2 = jnp.min(jnp.where(p2 == v2, iota, float(_N_EXPERTS)),
                 axis=0, keepdims=True)
    s = v1 + v2
    w2t = jnp.concatenate([v1 / s, v2 / s], axis=0)
    i2t = jnp.concatenate([i1, i2], axis=0).astype(jnp.int32)
    w_out_ref[...] = jnp.transpose(w2t)
    i_out_ref[...] = jnp.transpose(i2t)


def _gate_block(xa_ref, xb_ref, w_ref, b_ref, w_out_ref, i_out_ref):
    _gate_one(xa_ref, w_ref, b_ref,
              w_out_ref.at[pl.ds(0, _BLOCK_T), :],
              i_out_ref.at[pl.ds(0, _BLOCK_T), :])
    _gate_one(xb_ref, w_ref, b_ref,
              w_out_ref.at[pl.ds(_BLOCK_T, _BLOCK_T), :],
              i_out_ref.at[pl.ds(_BLOCK_T, _BLOCK_T), :])


def kernel(x, W, b):
    b2 = b.reshape(_N_EXPERTS, 1)
    grid = (_TOKENS // (2 * _BLOCK_T),)
    weights, indices = pl.pallas_call(
        _gate_block,
        grid=grid,
        in_specs=[
            pl.BlockSpec((_BLOCK_T, _DIM), lambda i: (2 * i, 0)),
            pl.BlockSpec((_BLOCK_T, _DIM), lambda i: (2 * i + 1, 0)),
            pl.BlockSpec((_N_EXPERTS, _DIM), lambda i: (0, 0)),
            pl.BlockSpec((_N_EXPERTS, 1), lambda i: (0, 0)),
        ],
        out_specs=[
            pl.BlockSpec((2 * _BLOCK_T, 2), lambda i: (i, 0)),
            pl.BlockSpec((2 * _BLOCK_T, 2), lambda i: (i, 0)),
        ],
        out_shape=[
            jax.ShapeDtypeStruct((_TOKENS, 2), jnp.float32),
            jax.ShapeDtypeStruct((_TOKENS, 2), jnp.int32),
        ],
        compiler_params=pltpu.CompilerParams(
            dimension_semantics=("arbitrary",),
        ),
    )(x, x, W, b2)
    return (weights, indices)
